# baseline (speedup 1.0000x reference)
import functools
import math

import jax
import jax.numpy as jnp
from jax import lax
from jax.experimental import pallas as pl
from jax.experimental.pallas import tpu as pltpu
from jax.experimental.pallas import tpu_sc as plsc

_N = 2048
_SIZE = 32000
_CONF = 0.9
_FILL = 0.1 / (_SIZE - 2)
_C = _CONF * math.log(_CONF) + 0.1 * math.log(_FILL)

_SC_ROWS = 512


_BR = 128
_GR = (_N - _SC_ROWS) // _BR
_ROW_OFF = _SC_ROWS // _BR


def _tc_body(tgt_ref, x_ref, out_ref):
    i = pl.program_id(0)

    x = x_ref[...]
    tgt = tgt_ref[0]
    live = tgt != 0
    rowcoef = jnp.where(live, -_FILL, 0.0).astype(jnp.float32)

    dense = lax.dot_general(
        rowcoef, x,
        dimension_numbers=(((0,), (0,)), ((), ())),
        preferred_element_type=jnp.float32,
    )

    teff = jnp.where(live, tgt, -1)
    col = lax.broadcasted_iota(jnp.int32, (_BR, _SIZE), 1)
    s_t = jnp.sum(jnp.where(col == teff, x, 0.0))

    c0 = jnp.sum(jnp.where(live, x_ref[:, 0:1], 0.0))
    cnt = jnp.sum(live.astype(jnp.float32))

    partial = (jnp.sum(dense) - (_CONF - _FILL) * s_t
               + _FILL * c0 + _C * cnt)

    @pl.when(i == 0)
    def _init():
        out_ref[0, 0] = 0.0

    out_ref[0, 0] += partial


def _tc_call(tgt3, x):
    return pl.pallas_call(
        _tc_body,
        grid=(_GR,),
        in_specs=[
            pl.BlockSpec((1, _BR, 1), lambda i: (i + _ROW_OFF, 0, 0)),
            pl.BlockSpec((_BR, _SIZE), lambda i: (i + _ROW_OFF, 0)),
        ],
        out_specs=pl.BlockSpec(
            (1, 1), lambda i: (0, 0), memory_space=pltpu.SMEM
        ),
        out_shape=jax.ShapeDtypeStruct((1, 1), jnp.float32),
    )(tgt3, x)


_NC = 2
_NS = 16
_L = 16
_NW = _NC * _NS
_RPW = _SC_ROWS // _NW
_NG = _RPW // 8
_CW = 3200
_NCH = _SIZE // _CW

_sc_mesh = plsc.VectorSubcoreMesh(core_axis_name="c", subcore_axis_name="s")


@functools.partial(
    pl.kernel,
    mesh=_sc_mesh,
    out_type=jax.ShapeDtypeStruct((_NW * _L,), jnp.float32),
    scratch_types=[
        pltpu.VMEM((_RPW,), jnp.int32),
        pltpu.VMEM((8, _CW), jnp.float32),
        pltpu.VMEM((8, _CW), jnp.float32),
        pltpu.VMEM((_L,), jnp.float32),
        pltpu.SemaphoreType.DMA,
        pltpu.SemaphoreType.DMA,
    ],
    compiler_params=pltpu.CompilerParams(
        use_tc_tiling_on_sc=True, needs_layout_passes=False
    ),
)
def _sc_kernel(x_hbm, tgt_hbm, out_hbm, tgt_v, buf0, buf1, acc_v, sem0, sem1):
    wid = lax.axis_index("s") * _NC + lax.axis_index("c")
    rbase = wid * _RPW
    pltpu.sync_copy(tgt_hbm.at[pl.ds(rbase, _RPW)], tgt_v)

    bufs = (buf0, buf1)
    sems = (sem0, sem1)
    lane8 = jnp.bitwise_and(lax.iota(jnp.int32, _L), 7)
    low8 = lax.iota(jnp.int32, _L) < 8

    def chunk_copy(g, c, b):
        return pltpu.make_async_copy(
            x_hbm.at[pl.ds(rbase + g * 8, 8), pl.ds(c * _CW, _CW)],
            bufs[b], sems[b],
        )

    total = jnp.zeros((_L,), jnp.float32)
    acc_t = jnp.zeros((_L,), jnp.float32)
    acc_0 = jnp.zeros((_L,), jnp.float32)

    chunk_copy(0, 0, 0).start()
    for g in range(_NG):
        t8 = plsc.load_gather(tgt_v, [g * 8 + lane8])
        rowsums = [jnp.zeros((_L,), jnp.float32) for _ in range(8)]
        for c in range(_NCH):
            b = (g * _NCH + c) % 2
            nxt = g * _NCH + c + 1
            if nxt < _NG * _NCH:
                chunk_copy(nxt // _NCH, nxt % _NCH, nxt % 2).start()
            chunk_copy(g, c, b).wait()
            buf = bufs[b]

            def body(l, carry):
                off = l * _L
                return tuple(
                    carry[s] + buf[s, pl.ds(off, _L)] for s in range(8)
                )
            rowsums = list(lax.fori_loop(0, _CW // _L, body, tuple(rowsums)))

            coff = t8 - c * _CW
            inrange = (coff >= 0) & (coff < _CW) & (t8 != 0) & low8
            cidx = jnp.clip(coff, 0, _CW - 1)
            vt = plsc.load_gather(buf, [lane8, cidx])
            acc_t = acc_t + jnp.where(inrange, vt, 0.0)

            if c == 0:
                v0 = plsc.load_gather(buf, [lane8, jnp.zeros((_L,), jnp.int32)])
                acc_0 = acc_0 + jnp.where((t8 != 0) & low8, v0, 0.0)

        for s in range(8):
            t_s = plsc.load_gather(tgt_v, [jnp.full((_L,), g * 8 + s, jnp.int32)])
            total = total + jnp.where(t_s != 0, rowsums[s], 0.0)

    tlive = (tgt_v[pl.ds(0, _L)] != 0)
    cnt = jnp.where(tlive, 1.0, 0.0)
    for k in range(1, _RPW // _L):
        cnt = cnt + jnp.where(tgt_v[pl.ds(k * _L, _L)] != 0, 1.0, 0.0)

    acc_v[...] = (-_FILL * total - (_CONF - _FILL) * acc_t
                  + _FILL * acc_0 + _C * cnt)
    pltpu.sync_copy(acc_v, out_hbm.at[pl.ds(wid * _L, _L)])


def kernel(x, target):
    tgt3 = target.reshape(_N // _BR, _BR, 1)
    dense = _tc_call(tgt3, x)
    sparse = _sc_kernel(x, target)
    return dense[0, 0] + jnp.sum(sparse)


# speedup vs baseline: 1.1548x; 1.1548x over previous
import math

import jax
import jax.numpy as jnp
from jax import lax
from jax.experimental import pallas as pl
from jax.experimental.pallas import tpu as pltpu

_N = 2048
_SIZE = 32000
_CONF = 0.9
_FILL = 0.1 / (_SIZE - 2)
_C = _CONF * math.log(_CONF) + 0.1 * math.log(_FILL)

_BR = 128
_GR = _N // _BR


def _body(tgt_ref, x_ref, out_ref):
    i = pl.program_id(0)

    x = x_ref[...]
    tgt = tgt_ref[0]
    live = tgt != 0
    rowcoef = jnp.where(live, -_FILL, 0.0).astype(jnp.float32)

    dense = lax.dot_general(
        rowcoef, x,
        dimension_numbers=(((0,), (0,)), ((), ())),
        preferred_element_type=jnp.float32,
    )

    teff = jnp.where(live, tgt, -1)
    col = lax.broadcasted_iota(jnp.int32, (_BR, _SIZE), 1)
    s_t = jnp.sum(jnp.where(col == teff, x, 0.0))

    c0 = jnp.sum(jnp.where(live, x_ref[:, 0:1], 0.0))
    cnt = jnp.sum(live.astype(jnp.float32))

    partial = (jnp.sum(dense) - (_CONF - _FILL) * s_t
               + _FILL * c0 + _C * cnt)

    @pl.when(i == 0)
    def _init():
        out_ref[0, 0] = 0.0

    out_ref[0, 0] += partial


def kernel(x, target):
    tgt3 = target.reshape(_GR, _BR, 1)
    out = pl.pallas_call(
        _body,
        grid=(_GR,),
        in_specs=[
            pl.BlockSpec((1, _BR, 1), lambda i: (i, 0, 0)),
            pl.BlockSpec((_BR, _SIZE), lambda i: (i, 0)),
        ],
        out_specs=pl.BlockSpec(
            (1, 1), lambda i: (0, 0), memory_space=pltpu.SMEM
        ),
        out_shape=jax.ShapeDtypeStruct((1, 1), jnp.float32),
    )(tgt3, x)
    return out[0, 0]
